# baseline (speedup 1.0000x reference)
import functools

import jax
import jax.numpy as jnp
from jax.experimental import pallas as pl
from jax.experimental.pallas import tpu as pltpu

_E = 8
_C = 192
_HID = 192
_B = 16
_S = 16
_P = _S * _S
_K = 9 * _C
_PAD = 128

_TAPS = [(k, 16 * (k // 3 - 1) + (k % 3 - 1), k % 3 - 1) for k in range(9)]


def _gate_kernel(meta_ref, w1_ref, b1_ref, w2_ref, b2_ref, top1_ref, bal_ref):
    meta = meta_ref[:]
    h = jax.lax.dot_general(meta, w1_ref[:], (((1,), (1,)), ((), ())),
                            preferred_element_type=jnp.float32)
    h = jnp.maximum(h + b1_ref[:], 0.0)
    logits = jax.lax.dot_general(h, w2_ref[:], (((1,), (1,)), ((), ())),
                                 preferred_element_type=jnp.float32)
    logits = logits + b2_ref[:]
    mx = jnp.max(logits, axis=1, keepdims=True)
    ex = jnp.exp(logits - mx)
    probs = ex / jnp.sum(ex, axis=1, keepdims=True)
    lane = jax.lax.broadcasted_iota(jnp.int32, (_B, _E), 1)
    is_max = logits == mx
    top1 = jnp.min(jnp.where(is_max, lane, _E), axis=1, keepdims=True)
    top1_ref[:] = top1
    imp = jnp.sum(probs, axis=0, keepdims=True)
    imp = imp / (jnp.sum(imp, axis=1, keepdims=True) + 1e-8)
    mean = jnp.sum(imp, axis=1, keepdims=True) / _E
    var = jnp.sum((imp - mean) ** 2, axis=1, keepdims=True) / (_E - 1)
    bal_ref[:, :] = jnp.sqrt(var)


def _moe_kernel(top1_ref, x_ref, w1_ref, w2_ref,
                bn1_g_ref, bn1_b_ref, bn2_g_ref, bn2_b_ref,
                out_ref, xcol_ref, h1_ref, h2_ref, hpad_ref,
                sc1_ref, sh1_ref, sc2_ref, sh2_ref):
    f32 = jnp.float32
    row = jax.lax.broadcasted_iota(jnp.int32, (_P, 1), 0)
    mask_m = (row % _S != 0).astype(f32)
    mask_p = (row % _S != _S - 1).astype(f32)

    def im2col(src):
        for k, off, dx in _TAPS:
            sl = src[_PAD + off:_PAD + off + _P, :]
            if dx == -1:
                sl = sl * mask_m
            elif dx == 1:
                sl = sl * mask_p
            xcol_ref[:, k * _C:(k + 1) * _C] = sl

    def affine(ssum, ssq, cnt, g_ref, b_ref, sc_ref, sh_ref):
        for e in range(_E):
            n = jnp.maximum(cnt[e:e + 1, :], 1.0) * _P
            m = ssum[e:e + 1, :] / n
            v = ssq[e:e + 1, :] / n - m * m
            sc = g_ref[e:e + 1, :] * jax.lax.rsqrt(v + 1e-5)
            sc_ref[e] = sc
            sh_ref[e] = b_ref[e:e + 1, :] - m * sc

    zero8 = jnp.zeros((_E, _C), f32)
    zcnt = jnp.zeros((_E, 1), f32)

    def pass1(b, carry):
        ssum, ssq, cnt = carry
        e = top1_ref[b, 0]
        im2col(x_ref[b])
        w = w1_ref[e]
        h = jnp.dot(xcol_ref[:, :], w, preferred_element_type=f32)
        h1_ref[b] = h
        oh = (jax.lax.broadcasted_iota(jnp.int32, (_E, 1), 0) == e).astype(f32)
        ssum = ssum + oh * jnp.sum(h, axis=0, keepdims=True)
        ssq = ssq + oh * jnp.sum(h * h, axis=0, keepdims=True)
        cnt = cnt + oh
        return ssum, ssq, cnt

    ssum1, ssq1, cnt = jax.lax.fori_loop(0, _B, pass1, (zero8, zero8, zcnt))
    affine(ssum1, ssq1, cnt, bn1_g_ref, bn1_b_ref, sc1_ref, sh1_ref)

    hpad_ref[:, :] = jnp.zeros((_P + 2 * _PAD, _C), f32)

    def pass2(b, carry):
        ssum, ssq = carry
        e = top1_ref[b, 0]
        hn = jnp.maximum(h1_ref[b] * sc1_ref[e] + sh1_ref[e], 0.0)
        hpad_ref[_PAD:_PAD + _P, :] = hn
        im2col(hpad_ref[:, :])
        w = w2_ref[e]
        h = jnp.dot(xcol_ref[:, :], w, preferred_element_type=f32)
        h2_ref[b] = h
        oh = (jax.lax.broadcasted_iota(jnp.int32, (_E, 1), 0) == e).astype(f32)
        ssum = ssum + oh * jnp.sum(h, axis=0, keepdims=True)
        ssq = ssq + oh * jnp.sum(h * h, axis=0, keepdims=True)
        return ssum, ssq

    ssum2, ssq2 = jax.lax.fori_loop(0, _B, pass2, (zero8, zero8))
    affine(ssum2, ssq2, cnt, bn2_g_ref, bn2_b_ref, sc2_ref, sh2_ref)

    def pass3(b, _):
        e = top1_ref[b, 0]
        out_ref[b] = jnp.maximum(h2_ref[b] * sc2_ref[e] + sh2_ref[e], 0.0)
        return 0

    jax.lax.fori_loop(0, _B, pass3, 0)


@functools.partial(jax.jit, static_argnames=("interpret",))
def kernel(moe_c4, meta, gate_w1, gate_b1, gate_w2, gate_b2, conv1_w, conv1_b,
           bn1_g, bn1_b, conv2_w, conv2_b, bn2_g, bn2_b, interpret=False):
    del conv1_b, conv2_b
    f32 = jnp.float32

    top1, bal = pl.pallas_call(
        _gate_kernel,
        out_shape=(jax.ShapeDtypeStruct((_B, 1), jnp.int32),
                   jax.ShapeDtypeStruct((1, 1), f32)),
        interpret=interpret,
    )(meta, gate_w1, gate_b1[None, :], gate_w2, gate_b2[None, :])

    x_t = moe_c4.reshape(_B, _C, _P).transpose(0, 2, 1)
    x_pad = jnp.pad(x_t, ((0, 0), (_PAD, _PAD), (0, 0)))
    w1t = conv1_w.transpose(0, 3, 4, 2, 1).reshape(_E, _K, _HID)
    w2t = conv2_w.transpose(0, 3, 4, 2, 1).reshape(_E, _K, _C)

    out = pl.pallas_call(
        _moe_kernel,
        out_shape=jax.ShapeDtypeStruct((_B, _P, _C), f32),
        in_specs=[
            pl.BlockSpec(memory_space=pltpu.SMEM),
            pl.BlockSpec(memory_space=pltpu.VMEM),
            pl.BlockSpec(memory_space=pltpu.VMEM),
            pl.BlockSpec(memory_space=pltpu.VMEM),
            pl.BlockSpec(memory_space=pltpu.VMEM),
            pl.BlockSpec(memory_space=pltpu.VMEM),
            pl.BlockSpec(memory_space=pltpu.VMEM),
            pl.BlockSpec(memory_space=pltpu.VMEM),
        ],
        scratch_shapes=[
            pltpu.VMEM((_P, _K), f32),
            pltpu.VMEM((_B, _P, _HID), f32),
            pltpu.VMEM((_B, _P, _C), f32),
            pltpu.VMEM((_P + 2 * _PAD, _C), f32),
            pltpu.VMEM((_E, 1, _HID), f32),
            pltpu.VMEM((_E, 1, _HID), f32),
            pltpu.VMEM((_E, 1, _C), f32),
            pltpu.VMEM((_E, 1, _C), f32),
        ],
        interpret=interpret,
    )(top1, x_pad, w1t, w2t, bn1_g, bn1_b, bn2_g, bn2_b)

    out = out.transpose(0, 2, 1).reshape(_B, _C, _S, _S)
    return out, bal[0, 0]


# speedup vs baseline: 1.5918x; 1.5918x over previous
import functools

import jax
import jax.numpy as jnp
from jax.experimental import pallas as pl
from jax.experimental.pallas import tpu as pltpu

_E = 8
_C = 192
_HID = 192
_B = 16
_S = 16
_P = _S * _S
_K = 9 * _C
_PAD = 128

_TAPS = [(k, 16 * (k // 3 - 1) + (k % 3 - 1), k % 3 - 1) for k in range(9)]


def _gate_kernel(meta_ref, w1_ref, b1_ref, w2_ref, b2_ref, top1_ref, bal_ref):
    meta = meta_ref[:]
    h = jax.lax.dot_general(meta, w1_ref[:], (((1,), (1,)), ((), ())),
                            preferred_element_type=jnp.float32)
    h = jnp.maximum(h + b1_ref[:], 0.0)
    logits = jax.lax.dot_general(h, w2_ref[:], (((1,), (1,)), ((), ())),
                                 preferred_element_type=jnp.float32)
    logits = logits + b2_ref[:]
    mx = jnp.max(logits, axis=1, keepdims=True)
    ex = jnp.exp(logits - mx)
    probs = ex / jnp.sum(ex, axis=1, keepdims=True)
    lane = jax.lax.broadcasted_iota(jnp.int32, (_B, _E), 1)
    is_max = logits == mx
    top1 = jnp.min(jnp.where(is_max, lane, _E), axis=1, keepdims=True)
    top1_ref[:] = top1
    imp = jnp.sum(probs, axis=0, keepdims=True)
    imp = imp / (jnp.sum(imp, axis=1, keepdims=True) + 1e-8)
    mean = jnp.sum(imp, axis=1, keepdims=True) / _E
    var = jnp.sum((imp - mean) ** 2, axis=1, keepdims=True) / (_E - 1)
    bal_ref[:, :] = jnp.sqrt(var)


def _moe_kernel(top1_ref, x_ref, w1_ref, w2_ref,
                bn1_g_ref, bn1_b_ref, bn2_g_ref, bn2_b_ref,
                out_ref, xa_ref, xb_ref, h1_ref, h2_ref, hpad_ref,
                sc1_ref, sh1_ref, sc2_ref, sh2_ref):
    f32 = jnp.float32
    lane = jax.lax.broadcasted_iota(jnp.int32, (1, _P), 1)
    mask_m = (lane % _S != 0).astype(f32)
    mask_p = (lane % _S != _S - 1).astype(f32)

    def im2col(src):
        for k, off, dx in _TAPS:
            sl = src[:, _PAD + off:_PAD + off + _P]
            if dx == -1:
                sl = sl * mask_m
            elif dx == 1:
                sl = sl * mask_p
            xa_ref[pl.Slice(k, _C, 9), :] = sl[:, :_P // 2]
            xb_ref[pl.Slice(k, _C, 9), :] = sl[:, _P // 2:]

    def conv(w):
        ha = jnp.dot(w, xa_ref[:, :], preferred_element_type=jnp.float32)
        hb = jnp.dot(w, xb_ref[:, :], preferred_element_type=jnp.float32)
        return jnp.concatenate([ha, hb], axis=1)

    def affine(ssum, ssq, cnt, g_ref, b_ref, sc_ref, sh_ref):
        for e in range(_E):
            n = jnp.maximum(cnt[:, e:e + 1], 1.0) * _P
            m = ssum[:, e:e + 1] / n
            v = ssq[:, e:e + 1] / n - m * m
            sc = g_ref[:, e:e + 1] * jax.lax.rsqrt(v + 1e-5)
            sc_ref[e] = sc
            sh_ref[e] = b_ref[:, e:e + 1] - m * sc

    zstat = jnp.zeros((_C, _E), f32)
    zcnt = jnp.zeros((1, _E), f32)

    def pass1(b, carry):
        ssum, ssq, cnt = carry
        e = top1_ref[b, 0]
        im2col(x_ref[b])
        h = conv(w1_ref[e])
        h1_ref[b] = h
        oh = (jax.lax.broadcasted_iota(jnp.int32, (1, _E), 1) == e).astype(f32)
        ssum = ssum + jnp.sum(h, axis=1, keepdims=True) * oh
        ssq = ssq + jnp.sum(h * h, axis=1, keepdims=True) * oh
        cnt = cnt + oh
        return ssum, ssq, cnt

    ssum1, ssq1, cnt = jax.lax.fori_loop(0, _B, pass1, (zstat, zstat, zcnt))
    affine(ssum1, ssq1, cnt, bn1_g_ref, bn1_b_ref, sc1_ref, sh1_ref)

    hpad_ref[:, :] = jnp.zeros((_C, _P + 2 * _PAD), f32)

    def pass2(b, carry):
        ssum, ssq = carry
        e = top1_ref[b, 0]
        hn = jnp.maximum(h1_ref[b] * sc1_ref[e] + sh1_ref[e], 0.0)
        hpad_ref[:, _PAD:_PAD + _P] = hn
        im2col(hpad_ref[:, :])
        h = conv(w2_ref[e])
        h2_ref[b] = h
        oh = (jax.lax.broadcasted_iota(jnp.int32, (1, _E), 1) == e).astype(f32)
        ssum = ssum + jnp.sum(h, axis=1, keepdims=True) * oh
        ssq = ssq + jnp.sum(h * h, axis=1, keepdims=True) * oh
        return ssum, ssq

    ssum2, ssq2 = jax.lax.fori_loop(0, _B, pass2, (zstat, zstat))
    affine(ssum2, ssq2, cnt, bn2_g_ref, bn2_b_ref, sc2_ref, sh2_ref)

    def pass3(b, _):
        e = top1_ref[b, 0]
        out_ref[b] = jnp.maximum(h2_ref[b] * sc2_ref[e] + sh2_ref[e], 0.0)
        return 0

    jax.lax.fori_loop(0, _B, pass3, 0)


@functools.partial(jax.jit, static_argnames=("interpret",))
def kernel(moe_c4, meta, gate_w1, gate_b1, gate_w2, gate_b2, conv1_w, conv1_b,
           bn1_g, bn1_b, conv2_w, conv2_b, bn2_g, bn2_b, interpret=False):
    del conv1_b, conv2_b
    f32 = jnp.float32

    top1, bal = pl.pallas_call(
        _gate_kernel,
        out_shape=(jax.ShapeDtypeStruct((_B, 1), jnp.int32),
                   jax.ShapeDtypeStruct((1, 1), f32)),
        interpret=interpret,
    )(meta, gate_w1, gate_b1[None, :], gate_w2, gate_b2[None, :])

    x_pad = jnp.pad(moe_c4.reshape(_B, _C, _P),
                    ((0, 0), (0, 0), (_PAD, _PAD)))
    w1n = conv1_w.reshape(_E, _HID, _K)
    w2n = conv2_w.reshape(_E, _C, _K)

    out = pl.pallas_call(
        _moe_kernel,
        out_shape=jax.ShapeDtypeStruct((_B, _C, _P), f32),
        in_specs=[
            pl.BlockSpec(memory_space=pltpu.SMEM),
            pl.BlockSpec(memory_space=pltpu.VMEM),
            pl.BlockSpec(memory_space=pltpu.VMEM),
            pl.BlockSpec(memory_space=pltpu.VMEM),
            pl.BlockSpec(memory_space=pltpu.VMEM),
            pl.BlockSpec(memory_space=pltpu.VMEM),
            pl.BlockSpec(memory_space=pltpu.VMEM),
            pl.BlockSpec(memory_space=pltpu.VMEM),
        ],
        scratch_shapes=[
            pltpu.VMEM((_K, _P // 2), f32),
            pltpu.VMEM((_K, _P // 2), f32),
            pltpu.VMEM((_B, _HID, _P), f32),
            pltpu.VMEM((_B, _C, _P), f32),
            pltpu.VMEM((_C, _P + 2 * _PAD), f32),
            pltpu.VMEM((_E, _HID, 1), f32),
            pltpu.VMEM((_E, _HID, 1), f32),
            pltpu.VMEM((_E, _C, 1), f32),
            pltpu.VMEM((_E, _C, 1), f32),
        ],
        interpret=interpret,
    )(top1, x_pad, w1n, w2n, bn1_g.T, bn1_b.T, bn2_g.T, bn2_b.T)

    return out.reshape(_B, _C, _S, _S), bal[0, 0]


# speedup vs baseline: 1.6174x; 1.0161x over previous
import functools

import jax
import jax.numpy as jnp
from jax.experimental import pallas as pl
from jax.experimental.pallas import tpu as pltpu

_E = 8
_C = 192
_HID = 192
_B = 16
_S = 16
_P = _S * _S
_K = 9 * _C
_PAD = 128

_TAPS = [(k, 16 * (k // 3 - 1) + (k % 3 - 1), k % 3 - 1) for k in range(9)]


def _gate_kernel(meta_ref, w1_ref, b1_ref, w2_ref, b2_ref, top1_ref, bal_ref):
    meta = meta_ref[:]
    h = jax.lax.dot_general(meta, w1_ref[:], (((1,), (1,)), ((), ())),
                            preferred_element_type=jnp.float32)
    h = jnp.maximum(h + b1_ref[:], 0.0)
    logits = jax.lax.dot_general(h, w2_ref[:], (((1,), (1,)), ((), ())),
                                 preferred_element_type=jnp.float32)
    logits = logits + b2_ref[:]
    mx = jnp.max(logits, axis=1, keepdims=True)
    ex = jnp.exp(logits - mx)
    probs = ex / jnp.sum(ex, axis=1, keepdims=True)
    lane = jax.lax.broadcasted_iota(jnp.int32, (_B, _E), 1)
    is_max = logits == mx
    top1 = jnp.min(jnp.where(is_max, lane, _E), axis=1, keepdims=True)
    top1_ref[:] = top1
    imp = jnp.sum(probs, axis=0, keepdims=True)
    imp = imp / (jnp.sum(imp, axis=1, keepdims=True) + 1e-8)
    mean = jnp.sum(imp, axis=1, keepdims=True) / _E
    var = jnp.sum((imp - mean) ** 2, axis=1, keepdims=True) / (_E - 1)
    bal_ref[:, :] = jnp.sqrt(var)


def _moe_kernel(top1_ref, x_ref, w1_ref, w2_ref,
                bn1_g_ref, bn1_b_ref, bn2_g_ref, bn2_b_ref,
                out_ref, xa_ref, xb_ref, h1_ref, h2_ref, hpad_ref,
                sc1_ref, sh1_ref, sc2_ref, sh2_ref):
    f32 = jnp.float32
    lane = jax.lax.broadcasted_iota(jnp.int32, (1, _P), 1)
    mask_m = (lane % _S != 0).astype(f32)
    mask_p = (lane % _S != _S - 1).astype(f32)

    def im2col(src):
        for k, off, dx in _TAPS:
            sl = src[:, _PAD + off:_PAD + off + _P]
            if dx == -1:
                sl = sl * mask_m
            elif dx == 1:
                sl = sl * mask_p
            xa_ref[pl.Slice(k, _C, 9), :] = sl[:, :_P // 2]
            xb_ref[pl.Slice(k, _C, 9), :] = sl[:, _P // 2:]

    def conv(w):
        ha = jnp.dot(w, xa_ref[:, :], preferred_element_type=jnp.float32)
        hb = jnp.dot(w, xb_ref[:, :], preferred_element_type=jnp.float32)
        return jnp.concatenate([ha, hb], axis=1)

    def affine(ssum, ssq, cnt, g_ref, b_ref, sc_ref, sh_ref):
        g_t = jnp.transpose(g_ref[:, :])
        b_t = jnp.transpose(b_ref[:, :])
        for e in range(_E):
            n = jnp.maximum(cnt[:, e:e + 1], 1.0) * _P
            m = ssum[:, e:e + 1] / n
            v = ssq[:, e:e + 1] / n - m * m
            sc = g_t[:, e:e + 1] * jax.lax.rsqrt(v + 1e-5)
            sc_ref[e] = sc
            sh_ref[e] = b_t[:, e:e + 1] - m * sc

    zstat = jnp.zeros((_C, _E), f32)
    zcnt = jnp.zeros((1, _E), f32)
    hpad_ref[:, :] = jnp.zeros((_C, _P + 2 * _PAD), f32)

    def pass1(b, carry):
        ssum, ssq, cnt = carry
        e = top1_ref[b, 0]
        hpad_ref[:, _PAD:_PAD + _P] = x_ref[b]
        im2col(hpad_ref[:, :])
        h = conv(w1_ref[e])
        h1_ref[b] = h
        oh = (jax.lax.broadcasted_iota(jnp.int32, (1, _E), 1) == e).astype(f32)
        ssum = ssum + jnp.sum(h, axis=1, keepdims=True) * oh
        ssq = ssq + jnp.sum(h * h, axis=1, keepdims=True) * oh
        cnt = cnt + oh
        return ssum, ssq, cnt

    ssum1, ssq1, cnt = jax.lax.fori_loop(0, _B, pass1, (zstat, zstat, zcnt))
    affine(ssum1, ssq1, cnt, bn1_g_ref, bn1_b_ref, sc1_ref, sh1_ref)

    def pass2(b, carry):
        ssum, ssq = carry
        e = top1_ref[b, 0]
        hn = jnp.maximum(h1_ref[b] * sc1_ref[e] + sh1_ref[e], 0.0)
        hpad_ref[:, _PAD:_PAD + _P] = hn
        im2col(hpad_ref[:, :])
        h = conv(w2_ref[e])
        h2_ref[b] = h
        oh = (jax.lax.broadcasted_iota(jnp.int32, (1, _E), 1) == e).astype(f32)
        ssum = ssum + jnp.sum(h, axis=1, keepdims=True) * oh
        ssq = ssq + jnp.sum(h * h, axis=1, keepdims=True) * oh
        return ssum, ssq

    ssum2, ssq2 = jax.lax.fori_loop(0, _B, pass2, (zstat, zstat))
    affine(ssum2, ssq2, cnt, bn2_g_ref, bn2_b_ref, sc2_ref, sh2_ref)

    def pass3(b, _):
        e = top1_ref[b, 0]
        out_ref[b] = jnp.maximum(h2_ref[b] * sc2_ref[e] + sh2_ref[e], 0.0)
        return 0

    jax.lax.fori_loop(0, _B, pass3, 0)


@functools.partial(jax.jit, static_argnames=("interpret",))
def kernel(moe_c4, meta, gate_w1, gate_b1, gate_w2, gate_b2, conv1_w, conv1_b,
           bn1_g, bn1_b, conv2_w, conv2_b, bn2_g, bn2_b, interpret=False):
    del conv1_b, conv2_b
    f32 = jnp.float32

    top1, bal = pl.pallas_call(
        _gate_kernel,
        out_shape=(jax.ShapeDtypeStruct((_B, 1), jnp.int32),
                   jax.ShapeDtypeStruct((1, 1), f32)),
        interpret=interpret,
    )(meta, gate_w1, gate_b1[None, :], gate_w2, gate_b2[None, :])

    x_flat = moe_c4.reshape(_B, _C, _P)
    w1n = conv1_w.reshape(_E, _HID, _K)
    w2n = conv2_w.reshape(_E, _C, _K)

    out = pl.pallas_call(
        _moe_kernel,
        out_shape=jax.ShapeDtypeStruct((_B, _C, _P), f32),
        in_specs=[
            pl.BlockSpec(memory_space=pltpu.SMEM),
            pl.BlockSpec(memory_space=pltpu.VMEM),
            pl.BlockSpec(memory_space=pltpu.VMEM),
            pl.BlockSpec(memory_space=pltpu.VMEM),
            pl.BlockSpec(memory_space=pltpu.VMEM),
            pl.BlockSpec(memory_space=pltpu.VMEM),
            pl.BlockSpec(memory_space=pltpu.VMEM),
            pl.BlockSpec(memory_space=pltpu.VMEM),
        ],
        scratch_shapes=[
            pltpu.VMEM((_K, _P // 2), f32),
            pltpu.VMEM((_K, _P // 2), f32),
            pltpu.VMEM((_B, _HID, _P), f32),
            pltpu.VMEM((_B, _C, _P), f32),
            pltpu.VMEM((_C, _P + 2 * _PAD), f32),
            pltpu.VMEM((_E, _HID, 1), f32),
            pltpu.VMEM((_E, _HID, 1), f32),
            pltpu.VMEM((_E, _C, 1), f32),
            pltpu.VMEM((_E, _C, 1), f32),
        ],
        interpret=interpret,
    )(top1, x_flat, w1n, w2n, bn1_g, bn1_b, bn2_g, bn2_b)

    return out.reshape(_B, _C, _S, _S), bal[0, 0]


# speedup vs baseline: 1.7014x; 1.0519x over previous
import functools

import jax
import jax.numpy as jnp
from jax.experimental import pallas as pl
from jax.experimental.pallas import tpu as pltpu

_E = 8
_C = 192
_HID = 192
_B = 16
_S = 16
_P = _S * _S
_K = 9 * _C
_PAD = 128

_TAPS = [(k, 16 * (k // 3 - 1) + (k % 3 - 1), k % 3 - 1) for k in range(9)]


def _gate_kernel(meta_ref, w1_ref, b1_ref, w2_ref, b2_ref, top1_ref, bal_ref):
    meta = meta_ref[:]
    h = jax.lax.dot_general(meta, w1_ref[:], (((1,), (1,)), ((), ())),
                            preferred_element_type=jnp.float32)
    h = jnp.maximum(h + b1_ref[:], 0.0)
    logits = jax.lax.dot_general(h, w2_ref[:], (((1,), (1,)), ((), ())),
                                 preferred_element_type=jnp.float32)
    logits = logits + b2_ref[:]
    mx = jnp.max(logits, axis=1, keepdims=True)
    ex = jnp.exp(logits - mx)
    probs = ex / jnp.sum(ex, axis=1, keepdims=True)
    lane = jax.lax.broadcasted_iota(jnp.int32, (_B, _E), 1)
    is_max = logits == mx
    top1 = jnp.min(jnp.where(is_max, lane, _E), axis=1, keepdims=True)
    top1_ref[:] = top1
    imp = jnp.sum(probs, axis=0, keepdims=True)
    imp = imp / (jnp.sum(imp, axis=1, keepdims=True) + 1e-8)
    mean = jnp.sum(imp, axis=1, keepdims=True) / _E
    var = jnp.sum((imp - mean) ** 2, axis=1, keepdims=True) / (_E - 1)
    bal_ref[:, :] = jnp.sqrt(var)


def _moe_kernel(top1_ref, x_ref, w1_ref, w2_ref,
                bn1_g_ref, bn1_b_ref, bn2_g_ref, bn2_b_ref,
                out_ref, xa_ref, xb_ref, h1_ref, h2_ref, hpad_ref,
                sc1_ref, sh1_ref, sc2_ref, sh2_ref, wbuf_ref, sem):
    f32 = jnp.float32
    lane = jax.lax.broadcasted_iota(jnp.int32, (1, _P), 1)
    mask_m = (lane % _S != 0).astype(f32)
    mask_p = (lane % _S != _S - 1).astype(f32)

    def fetch(w_ref, b, slot):
        e = top1_ref[b, 0]
        pltpu.make_async_copy(
            w_ref.at[:, e, :], wbuf_ref.at[slot], sem.at[slot]).start()

    def wait(w_ref, b, slot):
        e = top1_ref[b, 0]
        pltpu.make_async_copy(
            w_ref.at[:, e, :], wbuf_ref.at[slot], sem.at[slot]).wait()

    def im2col(src):
        for k, off, dx in _TAPS:
            sl = src[:, _PAD + off:_PAD + off + _P]
            if dx == -1:
                sl = sl * mask_m
            elif dx == 1:
                sl = sl * mask_p
            xa_ref[pl.Slice(k, _C, 9), :] = sl[:, :_P // 2]
            xb_ref[pl.Slice(k, _C, 9), :] = sl[:, _P // 2:]

    def conv(slot):
        w = wbuf_ref[slot]
        ha = jax.lax.dot_general(w, xa_ref[:, :], (((0,), (0,)), ((), ())),
                                 preferred_element_type=f32)
        hb = jax.lax.dot_general(w, xb_ref[:, :], (((0,), (0,)), ((), ())),
                                 preferred_element_type=f32)
        return jnp.concatenate([ha, hb], axis=1)

    def affine(ssum, ssq, cnt, g_ref, b_ref, sc_ref, sh_ref):
        g_t = jnp.transpose(g_ref[:, :])
        b_t = jnp.transpose(b_ref[:, :])
        for e in range(_E):
            n = jnp.maximum(cnt[:, e:e + 1], 1.0) * _P
            m = ssum[:, e:e + 1] / n
            v = ssq[:, e:e + 1] / n - m * m
            sc = g_t[:, e:e + 1] * jax.lax.rsqrt(v + 1e-5)
            sc_ref[e] = sc
            sh_ref[e] = b_t[:, e:e + 1] - m * sc

    zstat = jnp.zeros((_C, _E), f32)
    zcnt = jnp.zeros((1, _E), f32)
    hpad_ref[:, :] = jnp.zeros((_C, _P + 2 * _PAD), f32)
    fetch(w1_ref, 0, 0)

    def pass1(b, carry):
        ssum, ssq, cnt = carry
        e = top1_ref[b, 0]
        slot = jnp.bitwise_and(b, 1)
        jax.lax.cond(b + 1 < _B,
                     lambda: fetch(w1_ref, b + 1, 1 - slot), lambda: None)
        hpad_ref[:, _PAD:_PAD + _P] = jnp.transpose(x_ref[b])
        im2col(hpad_ref[:, :])
        wait(w1_ref, b, slot)
        h = conv(slot)
        h1_ref[b] = h
        oh = (jax.lax.broadcasted_iota(jnp.int32, (1, _E), 1) == e).astype(f32)
        ssum = ssum + jnp.sum(h, axis=1, keepdims=True) * oh
        ssq = ssq + jnp.sum(h * h, axis=1, keepdims=True) * oh
        cnt = cnt + oh
        return ssum, ssq, cnt

    ssum1, ssq1, cnt = jax.lax.fori_loop(0, _B, pass1, (zstat, zstat, zcnt))
    affine(ssum1, ssq1, cnt, bn1_g_ref, bn1_b_ref, sc1_ref, sh1_ref)

    fetch(w2_ref, 0, 0)

    def pass2(b, carry):
        ssum, ssq = carry
        e = top1_ref[b, 0]
        slot = jnp.bitwise_and(b, 1)
        jax.lax.cond(b + 1 < _B,
                     lambda: fetch(w2_ref, b + 1, 1 - slot), lambda: None)
        hn = jnp.maximum(h1_ref[b] * sc1_ref[e] + sh1_ref[e], 0.0)
        hpad_ref[:, _PAD:_PAD + _P] = hn
        im2col(hpad_ref[:, :])
        wait(w2_ref, b, slot)
        h = conv(slot)
        h2_ref[b] = h
        oh = (jax.lax.broadcasted_iota(jnp.int32, (1, _E), 1) == e).astype(f32)
        ssum = ssum + jnp.sum(h, axis=1, keepdims=True) * oh
        ssq = ssq + jnp.sum(h * h, axis=1, keepdims=True) * oh
        return ssum, ssq

    ssum2, ssq2 = jax.lax.fori_loop(0, _B, pass2, (zstat, zstat))
    affine(ssum2, ssq2, cnt, bn2_g_ref, bn2_b_ref, sc2_ref, sh2_ref)

    def pass3(b, _):
        e = top1_ref[b, 0]
        y = jnp.maximum(h2_ref[b] * sc2_ref[e] + sh2_ref[e], 0.0)
        out_ref[b] = jnp.transpose(y)
        return 0

    jax.lax.fori_loop(0, _B, pass3, 0)


@functools.partial(jax.jit, static_argnames=("interpret",))
def kernel(moe_c4, meta, gate_w1, gate_b1, gate_w2, gate_b2, conv1_w, conv1_b,
           bn1_g, bn1_b, conv2_w, conv2_b, bn2_g, bn2_b, interpret=False):
    del conv1_b, conv2_b
    f32 = jnp.float32

    top1, bal = pl.pallas_call(
        _gate_kernel,
        out_shape=(jax.ShapeDtypeStruct((_B, 1), jnp.int32),
                   jax.ShapeDtypeStruct((1, 1), f32)),
        interpret=interpret,
    )(meta, gate_w1, gate_b1[None, :], gate_w2, gate_b2[None, :])

    x_nat = moe_c4.transpose(0, 2, 3, 1).reshape(_B, _P, _C)
    w1p = conv1_w.transpose(2, 3, 4, 0, 1).reshape(_K, _E, _HID)
    w2p = conv2_w.transpose(2, 3, 4, 0, 1).reshape(_K, _E, _C)

    out = pl.pallas_call(
        _moe_kernel,
        out_shape=jax.ShapeDtypeStruct((_B, _P, _C), f32),
        in_specs=[
            pl.BlockSpec(memory_space=pltpu.SMEM),
            pl.BlockSpec(memory_space=pltpu.VMEM),
            pl.BlockSpec(memory_space=pltpu.MemorySpace.HBM),
            pl.BlockSpec(memory_space=pltpu.MemorySpace.HBM),
            pl.BlockSpec(memory_space=pltpu.VMEM),
            pl.BlockSpec(memory_space=pltpu.VMEM),
            pl.BlockSpec(memory_space=pltpu.VMEM),
            pl.BlockSpec(memory_space=pltpu.VMEM),
        ],
        scratch_shapes=[
            pltpu.VMEM((_K, _P // 2), f32),
            pltpu.VMEM((_K, _P // 2), f32),
            pltpu.VMEM((_B, _HID, _P), f32),
            pltpu.VMEM((_B, _C, _P), f32),
            pltpu.VMEM((_C, _P + 2 * _PAD), f32),
            pltpu.VMEM((_E, _HID, 1), f32),
            pltpu.VMEM((_E, _HID, 1), f32),
            pltpu.VMEM((_E, _C, 1), f32),
            pltpu.VMEM((_E, _C, 1), f32),
            pltpu.VMEM((2, _K, _C), f32),
            pltpu.SemaphoreType.DMA((2,)),
        ],
        interpret=interpret,
    )(top1, x_nat, w1p, w2p, bn1_g, bn1_b, bn2_g, bn2_b)

    return out.reshape(_B, _S, _S, _C).transpose(0, 3, 1, 2), bal[0, 0]


# speedup vs baseline: 4.6001x; 2.7037x over previous
import functools

import jax
import jax.numpy as jnp
from jax.experimental import pallas as pl
from jax.experimental.pallas import tpu as pltpu

_E = 8
_C = 192
_HID = 192
_B = 16
_S = 16
_P = _S * _S
_PAD = 128

_TAPS = [(k, 16 * (k // 3 - 1) + (k % 3 - 1), k % 3 - 1) for k in range(9)]


def _gate_kernel(meta_ref, w1_ref, b1_ref, w2_ref, b2_ref, top1_ref, bal_ref):
    meta = meta_ref[:]
    h = jax.lax.dot_general(meta, w1_ref[:], (((1,), (1,)), ((), ())),
                            preferred_element_type=jnp.float32)
    h = jnp.maximum(h + b1_ref[:], 0.0)
    logits = jax.lax.dot_general(h, w2_ref[:], (((1,), (1,)), ((), ())),
                                 preferred_element_type=jnp.float32)
    logits = logits + b2_ref[:]
    mx = jnp.max(logits, axis=1, keepdims=True)
    ex = jnp.exp(logits - mx)
    probs = ex / jnp.sum(ex, axis=1, keepdims=True)
    lane = jax.lax.broadcasted_iota(jnp.int32, (_B, _E), 1)
    is_max = logits == mx
    top1 = jnp.min(jnp.where(is_max, lane, _E), axis=1, keepdims=True)
    top1_ref[:] = top1
    imp = jnp.sum(probs, axis=0, keepdims=True)
    imp = imp / (jnp.sum(imp, axis=1, keepdims=True) + 1e-8)
    mean = jnp.sum(imp, axis=1, keepdims=True) / _E
    var = jnp.sum((imp - mean) ** 2, axis=1, keepdims=True) / (_E - 1)
    bal_ref[:, :] = jnp.sqrt(var)


def _moe_kernel(top1_ref, x_ref, w1_ref, w2_ref,
                bn1_g_ref, bn1_b_ref, bn2_g_ref, bn2_b_ref,
                out_ref, h1_ref, h2_ref, hpad_ref,
                sc1_ref, sh1_ref, sc2_ref, sh2_ref, wbuf_ref, sem):
    f32 = jnp.float32
    row = jax.lax.broadcasted_iota(jnp.int32, (_P, 1), 0)
    mask_m = (row % _S != 0).astype(f32)
    mask_p = (row % _S != _S - 1).astype(f32)

    def fetch(w_ref, b, slot):
        e = top1_ref[b, 0]
        pltpu.make_async_copy(
            w_ref.at[e], wbuf_ref.at[slot], sem.at[slot]).start()

    def wait(w_ref, b, slot):
        e = top1_ref[b, 0]
        pltpu.make_async_copy(
            w_ref.at[e], wbuf_ref.at[slot], sem.at[slot]).wait()

    def conv(slot):
        by_dx = {-1: None, 0: None, 1: None}
        for k, off, dx in _TAPS:
            xs = hpad_ref[_PAD + off:_PAD + off + _P, :]
            t = jax.lax.dot_general(xs, wbuf_ref[slot, k],
                                    (((1,), (1,)), ((), ())),
                                    preferred_element_type=f32)
            by_dx[dx] = t if by_dx[dx] is None else by_dx[dx] + t
        return by_dx[-1] * mask_m + by_dx[0] + by_dx[1] * mask_p

    def affine(ssum, ssq, cnt, g_ref, b_ref, sc_ref, sh_ref):
        for e in range(_E):
            n = jnp.maximum(cnt[e:e + 1, :], 1.0) * _P
            m = ssum[e:e + 1, :] / n
            v = ssq[e:e + 1, :] / n - m * m
            sc = g_ref[e:e + 1, :] * jax.lax.rsqrt(v + 1e-5)
            sc_ref[e] = sc
            sh_ref[e] = b_ref[e:e + 1, :] - m * sc

    zstat = jnp.zeros((_E, _C), f32)
    zcnt = jnp.zeros((_E, 1), f32)
    hpad_ref[:, :] = jnp.zeros((_P + 2 * _PAD, _C), f32)
    fetch(w1_ref, 0, 0)

    def pass1(b, carry):
        ssum, ssq, cnt = carry
        e = top1_ref[b, 0]
        slot = jnp.bitwise_and(b, 1)
        jax.lax.cond(b + 1 < _B,
                     lambda: fetch(w1_ref, b + 1, 1 - slot), lambda: None)
        hpad_ref[_PAD:_PAD + _P, :] = x_ref[b]
        wait(w1_ref, b, slot)
        h = conv(slot)
        h1_ref[b] = h
        oh = (jax.lax.broadcasted_iota(jnp.int32, (_E, 1), 0) == e).astype(f32)
        ssum = ssum + oh * jnp.sum(h, axis=0, keepdims=True)
        ssq = ssq + oh * jnp.sum(h * h, axis=0, keepdims=True)
        cnt = cnt + oh
        return ssum, ssq, cnt

    ssum1, ssq1, cnt = jax.lax.fori_loop(0, _B, pass1, (zstat, zstat, zcnt))
    affine(ssum1, ssq1, cnt, bn1_g_ref, bn1_b_ref, sc1_ref, sh1_ref)

    fetch(w2_ref, 0, 0)

    def pass2(b, carry):
        ssum, ssq = carry
        e = top1_ref[b, 0]
        slot = jnp.bitwise_and(b, 1)
        jax.lax.cond(b + 1 < _B,
                     lambda: fetch(w2_ref, b + 1, 1 - slot), lambda: None)
        hn = jnp.maximum(h1_ref[b] * sc1_ref[e] + sh1_ref[e], 0.0)
        hpad_ref[_PAD:_PAD + _P, :] = hn
        wait(w2_ref, b, slot)
        h = conv(slot)
        h2_ref[b] = h
        oh = (jax.lax.broadcasted_iota(jnp.int32, (_E, 1), 0) == e).astype(f32)
        ssum = ssum + oh * jnp.sum(h, axis=0, keepdims=True)
        ssq = ssq + oh * jnp.sum(h * h, axis=0, keepdims=True)
        return ssum, ssq

    ssum2, ssq2 = jax.lax.fori_loop(0, _B, pass2, (zstat, zstat))
    affine(ssum2, ssq2, cnt, bn2_g_ref, bn2_b_ref, sc2_ref, sh2_ref)

    def pass3(b, _):
        e = top1_ref[b, 0]
        out_ref[b] = jnp.maximum(h2_ref[b] * sc2_ref[e] + sh2_ref[e], 0.0)
        return 0

    jax.lax.fori_loop(0, _B, pass3, 0)


@functools.partial(jax.jit, static_argnames=("interpret",))
def kernel(moe_c4, meta, gate_w1, gate_b1, gate_w2, gate_b2, conv1_w, conv1_b,
           bn1_g, bn1_b, conv2_w, conv2_b, bn2_g, bn2_b, interpret=False):
    del conv1_b, conv2_b
    f32 = jnp.float32

    top1, bal = pl.pallas_call(
        _gate_kernel,
        out_shape=(jax.ShapeDtypeStruct((_B, 1), jnp.int32),
                   jax.ShapeDtypeStruct((1, 1), f32)),
        interpret=interpret,
    )(meta, gate_w1, gate_b1[None, :], gate_w2, gate_b2[None, :])

    x_nat = moe_c4.transpose(0, 2, 3, 1).reshape(_B, _P, _C)
    w1p = conv1_w.transpose(0, 3, 4, 1, 2).reshape(_E, 9, _HID, _C)
    w2p = conv2_w.transpose(0, 3, 4, 1, 2).reshape(_E, 9, _C, _HID)

    out = pl.pallas_call(
        _moe_kernel,
        out_shape=jax.ShapeDtypeStruct((_B, _P, _C), f32),
        in_specs=[
            pl.BlockSpec(memory_space=pltpu.SMEM),
            pl.BlockSpec(memory_space=pltpu.VMEM),
            pl.BlockSpec(memory_space=pltpu.MemorySpace.HBM),
            pl.BlockSpec(memory_space=pltpu.MemorySpace.HBM),
            pl.BlockSpec(memory_space=pltpu.VMEM),
            pl.BlockSpec(memory_space=pltpu.VMEM),
            pl.BlockSpec(memory_space=pltpu.VMEM),
            pl.BlockSpec(memory_space=pltpu.VMEM),
        ],
        scratch_shapes=[
            pltpu.VMEM((_B, _P, _HID), f32),
            pltpu.VMEM((_B, _P, _C), f32),
            pltpu.VMEM((_P + 2 * _PAD, _C), f32),
            pltpu.VMEM((_E, 1, _HID), f32),
            pltpu.VMEM((_E, 1, _HID), f32),
            pltpu.VMEM((_E, 1, _C), f32),
            pltpu.VMEM((_E, 1, _C), f32),
            pltpu.VMEM((2, 9, _HID, _C), f32),
            pltpu.SemaphoreType.DMA((2,)),
        ],
        interpret=interpret,
    )(top1, x_nat, w1p, w2p, bn1_g, bn1_b, bn2_g, bn2_b)

    return out.reshape(_B, _S, _S, _C).transpose(0, 3, 1, 2), bal[0, 0]
